# baseline (speedup 1.0000x reference)
import functools

import jax
import jax.numpy as jnp
from jax import lax
from jax.experimental import pallas as pl
from jax.experimental.pallas import tpu as pltpu
from jax.experimental.pallas import tpu_sc as plsc

T, N, D_IN, HID, HEADS = 2, 10000, 128, 64, 8
E = 160000
GRU_H = 256
N_CLASSES = 33

NC, NS, LANES = 2, 16, 16
NW = NC * NS
NPR = 40
NUM_RANGES = N // NPR
RPW = -(-NUM_RANGES // NW)
W = 64
NG = W // LANES
EP = E + 2000
OFFPAD = 272
LEAK = 0.2


def _mm_bias_body(x_ref, w_ref, b_ref, o_ref):
    o_ref[...] = (
        jnp.dot(x_ref[...], w_ref[...], preferred_element_type=jnp.float32)
        + b_ref[...]
    )


def _mm_bias(x, w, b, blk):
    m, k = x.shape
    f = w.shape[1]
    return pl.pallas_call(
        _mm_bias_body,
        grid=(m // blk,),
        in_specs=[
            pl.BlockSpec((blk, k), lambda i: (i, 0)),
            pl.BlockSpec((k, f), lambda i: (0, 0)),
            pl.BlockSpec((1, f), lambda i: (0, 0)),
        ],
        out_specs=pl.BlockSpec((blk, f), lambda i: (i, 0)),
        out_shape=jax.ShapeDtypeStruct((m, f), jnp.float32),
    )(x, w, b[None])


def _elu(v):
    return jnp.where(v > 0, v, jnp.exp(jnp.minimum(v, 0.0)) - 1.0)


def _stage2_body(g_ref, b1_ref, wl_ref, bl_ref, wr_ref, br_ref, xl_ref, xr_ref):
    h = _elu(g_ref[...] + b1_ref[...])
    xl_ref[...] = (
        jnp.dot(h, wl_ref[...], preferred_element_type=jnp.float32) + bl_ref[...]
    )
    xr_ref[...] = (
        jnp.dot(h, wr_ref[...], preferred_element_type=jnp.float32) + br_ref[...]
    )


def _stage2(g, b1, wl, bl, wr, br, blk):
    m, k = g.shape
    f = wl.shape[1]
    return pl.pallas_call(
        _stage2_body,
        grid=(m // blk,),
        in_specs=[
            pl.BlockSpec((blk, k), lambda i: (i, 0)),
            pl.BlockSpec((1, k), lambda i: (0, 0)),
            pl.BlockSpec((k, f), lambda i: (0, 0)),
            pl.BlockSpec((1, f), lambda i: (0, 0)),
            pl.BlockSpec((k, f), lambda i: (0, 0)),
            pl.BlockSpec((1, f), lambda i: (0, 0)),
        ],
        out_specs=[
            pl.BlockSpec((blk, f), lambda i: (i, 0)),
            pl.BlockSpec((blk, f), lambda i: (i, 0)),
        ],
        out_shape=[
            jax.ShapeDtypeStruct((m, f), jnp.float32),
            jax.ShapeDtypeStruct((m, f), jnp.float32),
        ],
    )(g, b1[None], wl, bl[None], wr, br[None])


def _gru_body(o0_ref, o1_ref, b2_ref, wif_ref, bif_ref, whf_ref, bhf_ref,
              wib_ref, bib_ref, whb_ref, bhb_ref, out_ref):
    i = pl.program_id(0)
    s0 = _elu(o0_ref[...] + b2_ref[...])
    s1 = _elu(o1_ref[...] + b2_ref[...])

    def dirstep(xa, xb, wi, bi, wh, bh):
        gia = jnp.dot(xa, wi, preferred_element_type=jnp.float32) + bi
        r = jax.nn.sigmoid(gia[:, :GRU_H] + bh[:, :GRU_H])
        z = jax.nn.sigmoid(gia[:, GRU_H:2 * GRU_H] + bh[:, GRU_H:2 * GRU_H])
        ng = jnp.tanh(gia[:, 2 * GRU_H:] + r * bh[:, 2 * GRU_H:])
        h1 = (1.0 - z) * ng
        gib = jnp.dot(xb, wi, preferred_element_type=jnp.float32) + bi
        gh = jnp.dot(h1, wh, preferred_element_type=jnp.float32) + bh
        r2 = jax.nn.sigmoid(gib[:, :GRU_H] + gh[:, :GRU_H])
        z2 = jax.nn.sigmoid(gib[:, GRU_H:2 * GRU_H] + gh[:, GRU_H:2 * GRU_H])
        n2 = jnp.tanh(gib[:, 2 * GRU_H:] + r2 * gh[:, 2 * GRU_H:])
        return (1.0 - z2) * n2 + z2 * h1

    hf = dirstep(s0, s1, wif_ref[...], bif_ref[...], whf_ref[...], bhf_ref[...])
    hb = dirstep(s1, s0, wib_ref[...], bib_ref[...], whb_ref[...], bhb_ref[...])
    psum = jnp.concatenate(
        [jnp.sum(hf, axis=0, keepdims=True), jnp.sum(hb, axis=0, keepdims=True)],
        axis=1,
    )

    @pl.when(i == 0)
    def _():
        out_ref[...] = jnp.zeros_like(out_ref)

    out_ref[...] += psum


def _gru_mean(o0, o1, b2, gp, blk):
    f = o0.shape[1]
    wif = gp["fwd"]["W_ih"].T
    whf = gp["fwd"]["W_hh"].T
    wib = gp["bwd"]["W_ih"].T
    whb = gp["bwd"]["W_hh"].T
    g3 = 3 * GRU_H
    return pl.pallas_call(
        _gru_body,
        grid=(N // blk,),
        in_specs=[
            pl.BlockSpec((blk, f), lambda i: (i, 0)),
            pl.BlockSpec((blk, f), lambda i: (i, 0)),
            pl.BlockSpec((1, f), lambda i: (0, 0)),
            pl.BlockSpec((f, g3), lambda i: (0, 0)),
            pl.BlockSpec((1, g3), lambda i: (0, 0)),
            pl.BlockSpec((GRU_H, g3), lambda i: (0, 0)),
            pl.BlockSpec((1, g3), lambda i: (0, 0)),
            pl.BlockSpec((f, g3), lambda i: (0, 0)),
            pl.BlockSpec((1, g3), lambda i: (0, 0)),
            pl.BlockSpec((GRU_H, g3), lambda i: (0, 0)),
            pl.BlockSpec((1, g3), lambda i: (0, 0)),
        ],
        out_specs=pl.BlockSpec((1, 2 * GRU_H), lambda i: (0, 0)),
        out_shape=jax.ShapeDtypeStruct((1, 2 * GRU_H), jnp.float32),
    )(o0, o1, b2[None], wif, gp["fwd"]["b_ih"][None], whf,
      gp["fwd"]["b_hh"][None], wib, gp["bwd"]["b_ih"][None], whb,
      gp["bwd"]["b_hh"][None])


def _fc_body(s_ref, w_ref, b_ref, o_ref):
    g = s_ref[...] * (1.0 / N)
    o_ref[...] = (
        jnp.dot(g, w_ref[...], preferred_element_type=jnp.float32) + b_ref[...]
    )


def _fc(s, w, b):
    return pl.pallas_call(
        _fc_body,
        out_shape=jax.ShapeDtypeStruct((1, N_CLASSES), jnp.float32),
    )(s, w, b[None])


def _sc_body(xl, xr, eet, srcs, dsts, offs, attf, out,
             xrbuf, outbuf, xjbuf, eebuf, srcb, dstb, den, parts, exs,
             attb, offb, sem1, sem2, sem3, *, F, H):
    CPH = (F // H) // LANES
    NCH = F // LANES
    cid = lax.axis_index("c")
    sid = lax.axis_index("s")
    wid = sid * NC + cid
    pltpu.sync_copy(offs, offb)
    pltpu.sync_copy(attf, attb)
    zero = jnp.zeros((LANES,), jnp.float32)
    iot = lax.iota(jnp.int32, LANES)

    def range_body(rr, _):
        rid = wid + rr * NW

        @pl.when(rid < NUM_RANGES)
        def _():
            n0 = rid * NPR
            es = offb[pl.ds(rid, LANES)][0]
            ed = offb[pl.ds(rid + 1, LANES)][0]
            a0 = (es // 8) * 8
            nstrips = (ed - a0 + W - 1) // W

            def zrow(i, _):
                for v in range(NCH):
                    outbuf[i, pl.ds(LANES * v, LANES)] = zero
                return 0

            lax.fori_loop(0, NPR, zrow, 0)
            for j in range((H * NPR + LANES) // LANES):
                den[pl.ds(LANES * j, LANES)] = zero

            pltpu.sync_copy(xr.at[pl.ds(n0, NPR)], xrbuf)

            def strip(k, _):
                base = a0 + W * k
                c1 = pltpu.async_copy(srcs.at[pl.ds(base, W)], srcb, sem1)
                c2 = pltpu.async_copy(
                    dsts.at[pl.ds(base, W)], dstb.at[pl.ds(0, W)], sem2
                )
                c3 = pltpu.async_copy(eet.at[pl.ds(base, W)], eebuf, sem3)
                c1.wait()
                pltpu.async_copy(xl.at[srcb], xjbuf, sem1).wait()
                c2.wait()
                c3.wait()

                @plsc.parallel_loop(0, W, unroll=4)
                def edge_alpha(e):
                    d = jnp.clip(dstb[pl.ds(e, LANES)][0] - n0, 0, NPR - 1)
                    acc = [zero] * H
                    for v in range(NCH):
                        h = v // CPH
                        sl = pl.ds(LANES * v, LANES)
                        s = xjbuf[e, sl] + xrbuf[d, sl] + eebuf[e, sl]
                        m = jnp.maximum(s, LEAK * s)
                        acc[h] = acc[h] + m * attb[sl]
                    for h in range(H):
                        parts[pl.ds((e * H + h) * LANES, LANES)] = acc[h]

                for g in range(NG):
                    gid = base + g * LANES + iot
                    valid = (gid >= es) & (gid < ed)
                    dstl = jnp.clip(
                        dstb[pl.ds(g * LANES, LANES)] - n0, 0, NPR - 1
                    )
                    for h in range(H):
                        gs = [
                            plsc.load_gather(
                                parts,
                                [(g * LANES + iot) * (H * LANES)
                                 + (h * LANES + l)],
                            )
                            for l in range(LANES)
                        ]
                        while len(gs) > 1:
                            gs = [
                                gs[i] + gs[i + 1]
                                for i in range(0, len(gs) - 1, 2)
                            ] + ([gs[-1]] if len(gs) % 2 else [])
                        exh = jnp.where(valid, jnp.exp(gs[0]), 0.0)
                        plsc.addupdate_scatter(den, [h * NPR + dstl], exh)
                        exs[pl.ds(h * W + g * LANES, LANES)] = exh

                @plsc.parallel_loop(0, W, unroll=4)
                def edge_acc(e):
                    g = base + e

                    @pl.when((g >= es) & (g < ed))
                    def _():
                        d = jnp.clip(dstb[pl.ds(e, LANES)][0] - n0, 0, NPR - 1)
                        for h in range(H):
                            a = exs[pl.ds(h * W + e, LANES)][0]
                            for u in range(CPH):
                                sl = pl.ds(LANES * (h * CPH + u), LANES)
                                plsc.addupdate(
                                    outbuf.at[d, sl], a * xjbuf[e, sl]
                                )
                return 0

            lax.fori_loop(0, nstrips, strip, 0)

            def norm_row(i, _):
                for h in range(H):
                    dsc = den[pl.ds(h * NPR + i, LANES)][0]
                    rv = 1.0 / (jnp.full((LANES,), dsc) + 1e-16)
                    for u in range(CPH):
                        sl = pl.ds(LANES * (h * CPH + u), LANES)
                        outbuf[i, sl] = outbuf[i, sl] * rv
                return 0

            lax.fori_loop(0, NPR, norm_row, 0)
            pltpu.sync_copy(outbuf, out.at[pl.ds(n0, NPR)])
        return 0

    lax.fori_loop(0, RPW, range_body, 0)


def _sc_gat(xl, xr, eet, srcs, dsts, offs, attf, F, H):
    mesh = plsc.VectorSubcoreMesh(core_axis_name="c", subcore_axis_name="s")
    body = functools.partial(_sc_body, F=F, H=H)
    return pl.kernel(
        body,
        out_type=jax.ShapeDtypeStruct((N, F), jnp.float32),
        mesh=mesh,
        compiler_params=pltpu.CompilerParams(needs_layout_passes=False),
        scratch_types=[
            pltpu.VMEM((NPR, F), jnp.float32),
            pltpu.VMEM((NPR, F), jnp.float32),
            pltpu.VMEM((W, F), jnp.float32),
            pltpu.VMEM((W, F), jnp.float32),
            pltpu.VMEM((W,), jnp.int32),
            pltpu.VMEM((W + LANES,), jnp.int32),
            pltpu.VMEM((H * NPR + LANES,), jnp.float32),
            pltpu.VMEM((W * H * LANES,), jnp.float32),
            pltpu.VMEM((H * W + LANES,), jnp.float32),
            pltpu.VMEM((F,), jnp.float32),
            pltpu.VMEM((OFFPAD,), jnp.int32),
            pltpu.SemaphoreType.DMA,
            pltpu.SemaphoreType.DMA,
            pltpu.SemaphoreType.DMA,
        ],
    )(xl, xr, eet, srcs, dsts, offs, attf)


def kernel(x, edge_index, edge_attr, params):
    src, dst = edge_index[0], edge_index[1]
    order = jnp.argsort(dst)
    src_s = jnp.concatenate([src[order], jnp.zeros((EP - E,), jnp.int32)])
    dst_s = jnp.concatenate([dst[order], jnp.zeros((EP - E,), jnp.int32)])
    ea_s = jnp.concatenate(
        [edge_attr[order], jnp.zeros((EP - E, edge_attr.shape[1]), jnp.float32)]
    )
    bounds = jnp.arange(0, N + 1, NPR, dtype=jnp.int32)
    offs = jnp.searchsorted(dst_s[:E], bounds).astype(jnp.int32)
    offs = jnp.concatenate(
        [offs, jnp.full((OFFPAD - NUM_RANGES - 1,), E, jnp.int32)]
    )

    p1, p2 = params["gat1"], params["gat2"]
    f1, f2 = HEADS * HID, HID

    x2 = x.reshape(T * N, D_IN)
    xl1 = _mm_bias(x2, p1["Wl"], p1["bl"], 1000).reshape(T, N, f1)
    xr1 = _mm_bias(x2, p1["Wr"], p1["br"], 1000).reshape(T, N, f1)
    ee1 = _mm_bias(ea_s, p1["We"], jnp.zeros((f1,), jnp.float32), 2000)
    att1 = p1["att"].reshape(-1)
    g1 = [
        _sc_gat(xl1[t], xr1[t], ee1, src_s, dst_s, offs, att1, f1, HEADS)
        for t in range(T)
    ]

    g1all = jnp.concatenate(g1, axis=0)
    xl2, xr2 = _stage2(
        g1all, p1["bias"], p2["Wl"], p2["bl"], p2["Wr"], p2["br"], 1000
    )
    xl2p = jnp.concatenate([xl2[:N], xl2[N:]], axis=1)
    xr2p = jnp.concatenate([xr2[:N], xr2[N:]], axis=1)
    we2p = jnp.concatenate([p2["We"], p2["We"]], axis=1)
    ee2 = _mm_bias(ea_s, we2p, jnp.zeros((2 * f2,), jnp.float32), 2000)
    att2 = jnp.concatenate([p2["att"].reshape(-1)] * 2)
    g2p = _sc_gat(xl2p, xr2p, ee2, src_s, dst_s, offs, att2, 2 * f2, 2)

    s = _gru_mean(g2p[:, :f2], g2p[:, f2:], p2["bias"], params["gru"][0], 1000)
    return _fc(s, params["fc"]["W"], params["fc"]["b"])


# speedup vs baseline: 1.0681x; 1.0681x over previous
import functools

import jax
import jax.numpy as jnp
from jax import lax
from jax.experimental import pallas as pl
from jax.experimental.pallas import tpu as pltpu
from jax.experimental.pallas import tpu_sc as plsc

T, N, D_IN, HID, HEADS = 2, 10000, 128, 64, 8
E = 160000
GRU_H = 256
N_CLASSES = 33

NC, NS, LANES = 2, 16, 16
NW = NC * NS
NPR = 40
NUM_RANGES = N // NPR
RPW = -(-NUM_RANGES // NW)
W = 32
NG = W // LANES
EP = E + 2000
OFFPAD = 272
LEAK = 0.2


def _mm_bias_body(x_ref, w_ref, b_ref, o_ref):
    o_ref[...] = (
        jnp.dot(x_ref[...], w_ref[...], preferred_element_type=jnp.float32)
        + b_ref[...]
    )


def _mm_bias(x, w, b, blk):
    m, k = x.shape
    f = w.shape[1]
    return pl.pallas_call(
        _mm_bias_body,
        grid=(m // blk,),
        in_specs=[
            pl.BlockSpec((blk, k), lambda i: (i, 0)),
            pl.BlockSpec((k, f), lambda i: (0, 0)),
            pl.BlockSpec((1, f), lambda i: (0, 0)),
        ],
        out_specs=pl.BlockSpec((blk, f), lambda i: (i, 0)),
        out_shape=jax.ShapeDtypeStruct((m, f), jnp.float32),
    )(x, w, b[None])


def _elu(v):
    return jnp.where(v > 0, v, jnp.exp(jnp.minimum(v, 0.0)) - 1.0)


def _stage2_body(g_ref, b1_ref, wl_ref, bl_ref, wr_ref, br_ref, xl_ref, xr_ref):
    h = _elu(g_ref[...] + b1_ref[...])
    xl_ref[...] = (
        jnp.dot(h, wl_ref[...], preferred_element_type=jnp.float32) + bl_ref[...]
    )
    xr_ref[...] = (
        jnp.dot(h, wr_ref[...], preferred_element_type=jnp.float32) + br_ref[...]
    )


def _stage2(g, b1, wl, bl, wr, br, blk):
    m, k = g.shape
    f = wl.shape[1]
    return pl.pallas_call(
        _stage2_body,
        grid=(m // blk,),
        in_specs=[
            pl.BlockSpec((blk, k), lambda i: (i, 0)),
            pl.BlockSpec((1, k), lambda i: (0, 0)),
            pl.BlockSpec((k, f), lambda i: (0, 0)),
            pl.BlockSpec((1, f), lambda i: (0, 0)),
            pl.BlockSpec((k, f), lambda i: (0, 0)),
            pl.BlockSpec((1, f), lambda i: (0, 0)),
        ],
        out_specs=[
            pl.BlockSpec((blk, f), lambda i: (i, 0)),
            pl.BlockSpec((blk, f), lambda i: (i, 0)),
        ],
        out_shape=[
            jax.ShapeDtypeStruct((m, f), jnp.float32),
            jax.ShapeDtypeStruct((m, f), jnp.float32),
        ],
    )(g, b1[None], wl, bl[None], wr, br[None])


def _gru_body(o0_ref, o1_ref, b2_ref, wif_ref, bif_ref, whf_ref, bhf_ref,
              wib_ref, bib_ref, whb_ref, bhb_ref, out_ref):
    i = pl.program_id(0)
    s0 = _elu(o0_ref[...] + b2_ref[...])
    s1 = _elu(o1_ref[...] + b2_ref[...])

    def dirstep(xa, xb, wi, bi, wh, bh):
        gia = jnp.dot(xa, wi, preferred_element_type=jnp.float32) + bi
        r = jax.nn.sigmoid(gia[:, :GRU_H] + bh[:, :GRU_H])
        z = jax.nn.sigmoid(gia[:, GRU_H:2 * GRU_H] + bh[:, GRU_H:2 * GRU_H])
        ng = jnp.tanh(gia[:, 2 * GRU_H:] + r * bh[:, 2 * GRU_H:])
        h1 = (1.0 - z) * ng
        gib = jnp.dot(xb, wi, preferred_element_type=jnp.float32) + bi
        gh = jnp.dot(h1, wh, preferred_element_type=jnp.float32) + bh
        r2 = jax.nn.sigmoid(gib[:, :GRU_H] + gh[:, :GRU_H])
        z2 = jax.nn.sigmoid(gib[:, GRU_H:2 * GRU_H] + gh[:, GRU_H:2 * GRU_H])
        n2 = jnp.tanh(gib[:, 2 * GRU_H:] + r2 * gh[:, 2 * GRU_H:])
        return (1.0 - z2) * n2 + z2 * h1

    hf = dirstep(s0, s1, wif_ref[...], bif_ref[...], whf_ref[...], bhf_ref[...])
    hb = dirstep(s1, s0, wib_ref[...], bib_ref[...], whb_ref[...], bhb_ref[...])
    psum = jnp.concatenate(
        [jnp.sum(hf, axis=0, keepdims=True), jnp.sum(hb, axis=0, keepdims=True)],
        axis=1,
    )

    @pl.when(i == 0)
    def _():
        out_ref[...] = jnp.zeros_like(out_ref)

    out_ref[...] += psum


def _gru_mean(o0, o1, b2, gp, blk):
    f = o0.shape[1]
    wif = gp["fwd"]["W_ih"].T
    whf = gp["fwd"]["W_hh"].T
    wib = gp["bwd"]["W_ih"].T
    whb = gp["bwd"]["W_hh"].T
    g3 = 3 * GRU_H
    return pl.pallas_call(
        _gru_body,
        grid=(N // blk,),
        in_specs=[
            pl.BlockSpec((blk, f), lambda i: (i, 0)),
            pl.BlockSpec((blk, f), lambda i: (i, 0)),
            pl.BlockSpec((1, f), lambda i: (0, 0)),
            pl.BlockSpec((f, g3), lambda i: (0, 0)),
            pl.BlockSpec((1, g3), lambda i: (0, 0)),
            pl.BlockSpec((GRU_H, g3), lambda i: (0, 0)),
            pl.BlockSpec((1, g3), lambda i: (0, 0)),
            pl.BlockSpec((f, g3), lambda i: (0, 0)),
            pl.BlockSpec((1, g3), lambda i: (0, 0)),
            pl.BlockSpec((GRU_H, g3), lambda i: (0, 0)),
            pl.BlockSpec((1, g3), lambda i: (0, 0)),
        ],
        out_specs=pl.BlockSpec((1, 2 * GRU_H), lambda i: (0, 0)),
        out_shape=jax.ShapeDtypeStruct((1, 2 * GRU_H), jnp.float32),
    )(o0, o1, b2[None], wif, gp["fwd"]["b_ih"][None], whf,
      gp["fwd"]["b_hh"][None], wib, gp["bwd"]["b_ih"][None], whb,
      gp["bwd"]["b_hh"][None])


def _fc_body(s_ref, w_ref, b_ref, o_ref):
    g = s_ref[...] * (1.0 / N)
    o_ref[...] = (
        jnp.dot(g, w_ref[...], preferred_element_type=jnp.float32) + b_ref[...]
    )


def _fc(s, w, b):
    return pl.pallas_call(
        _fc_body,
        out_shape=jax.ShapeDtypeStruct((1, N_CLASSES), jnp.float32),
    )(s, w, b[None])


def _sc_body(xl, xr, eet, srcs, dsts, offs, attf, out,
             xrbuf, outbuf, xjb0, xjb1, eeb0, eeb1, srb0, srb1, dsb0, dsb1,
             den, parts, exs, attb, offb, s1, s2, s3, sxj, *, F, H):
    xjb, eeb, srb, dsb = (xjb0, xjb1), (eeb0, eeb1), (srb0, srb1), (dsb0, dsb1)
    CPH = (F // H) // LANES
    NCH = F // LANES
    cid = lax.axis_index("c")
    sid = lax.axis_index("s")
    wid = sid * NC + cid
    pltpu.sync_copy(offs, offb)
    pltpu.sync_copy(attf, attb)
    zero = jnp.zeros((LANES,), jnp.float32)
    iot = lax.iota(jnp.int32, LANES)

    def range_body(rr, _):
        rid = wid + rr * NW

        @pl.when(rid < NUM_RANGES)
        def _():
            n0 = rid * NPR
            es = offb[pl.ds(rid, LANES)][0]
            ed = offb[pl.ds(rid + 1, LANES)][0]
            a0 = (es // 8) * 8
            nstrips = (ed - a0 + W - 1) // W

            def zrow(i, _):
                for v in range(NCH):
                    outbuf[i, pl.ds(LANES * v, LANES)] = zero
                return 0

            lax.fori_loop(0, NPR, zrow, 0)
            for j in range((H * NPR + LANES) // LANES):
                den[pl.ds(LANES * j, LANES)] = zero

            pltpu.sync_copy(xr.at[pl.ds(n0, NPR)], xrbuf)

            def load_idx(k, b):
                base = a0 + W * k
                c1 = pltpu.async_copy(srcs.at[pl.ds(base, W)], srb[b], s1)
                c2 = pltpu.async_copy(
                    dsts.at[pl.ds(base, W)], dsb[b].at[pl.ds(0, W)], s2
                )
                c3 = pltpu.async_copy(eet.at[pl.ds(base, W)], eeb[b], s3)
                c1.wait()
                pltpu.async_copy(xl.at[srb[b]], xjb[b], sxj)
                c2.wait()
                c3.wait()

            @pl.when(nstrips > 0)
            def _():
                load_idx(0, 0)

            def compute_strip(k, b):
                base = a0 + W * k
                xjbuf, eebuf, dstb = xjb[b], eeb[b], dsb[b]

                @plsc.parallel_loop(0, W, unroll=2)
                def edge_alpha(e):
                    d = jnp.clip(dstb[pl.ds(e, LANES)][0] - n0, 0, NPR - 1)
                    acc = [zero] * H
                    for v in range(NCH):
                        h = v // CPH
                        sl = pl.ds(LANES * v, LANES)
                        s = xjbuf[e, sl] + xrbuf[d, sl] + eebuf[e, sl]
                        m = jnp.maximum(s, LEAK * s)
                        acc[h] = acc[h] + m * attb[sl]
                    for h in range(H):
                        parts[pl.ds((e * H + h) * LANES, LANES)] = acc[h]

                for g in range(NG):
                    gid = base + g * LANES + iot
                    valid = (gid >= es) & (gid < ed)
                    dstl = jnp.clip(
                        dstb[pl.ds(g * LANES, LANES)] - n0, 0, NPR - 1
                    )
                    for h in range(H):
                        gs = [
                            plsc.load_gather(
                                parts,
                                [(g * LANES + iot) * (H * LANES)
                                 + (h * LANES + l)],
                            )
                            for l in range(LANES)
                        ]
                        while len(gs) > 1:
                            gs = [
                                gs[i] + gs[i + 1]
                                for i in range(0, len(gs) - 1, 2)
                            ] + ([gs[-1]] if len(gs) % 2 else [])
                        exh = jnp.where(valid, jnp.exp(gs[0]), 0.0)
                        plsc.addupdate_scatter(den, [h * NPR + dstl], exh)
                        exs[pl.ds(h * W + g * LANES, LANES)] = exh

                @plsc.parallel_loop(0, W, unroll=2)
                def edge_acc(e):
                    g = base + e

                    @pl.when((g >= es) & (g < ed))
                    def _():
                        d = jnp.clip(dstb[pl.ds(e, LANES)][0] - n0, 0, NPR - 1)
                        for h in range(H):
                            a = exs[pl.ds(h * W + e, LANES)][0]
                            for u in range(CPH):
                                sl = pl.ds(LANES * (h * CPH + u), LANES)
                                plsc.addupdate(
                                    outbuf.at[d, sl], a * xjbuf[e, sl]
                                )

            def outer(ko, _):
                for b in (0, 1):
                    k = 2 * ko + b

                    @pl.when(k < nstrips)
                    def _(k=k, b=b):
                        pltpu.make_async_copy(
                            xl.at[pl.ds(0, W)], xjb[b], sxj
                        ).wait()

                        @pl.when(k + 1 < nstrips)
                        def _():
                            load_idx(k + 1, 1 - b)

                        compute_strip(k, b)
                return 0

            lax.fori_loop(0, (nstrips + 1) // 2, outer, 0)

            def norm_row(i, _):
                for h in range(H):
                    dsc = den[pl.ds(h * NPR + i, LANES)][0]
                    rv = 1.0 / (jnp.full((LANES,), dsc) + 1e-16)
                    for u in range(CPH):
                        sl = pl.ds(LANES * (h * CPH + u), LANES)
                        outbuf[i, sl] = outbuf[i, sl] * rv
                return 0

            lax.fori_loop(0, NPR, norm_row, 0)
            pltpu.sync_copy(outbuf, out.at[pl.ds(n0, NPR)])
        return 0

    lax.fori_loop(0, RPW, range_body, 0)


def _sc_gat(xl, xr, eet, srcs, dsts, offs, attf, F, H):
    mesh = plsc.VectorSubcoreMesh(core_axis_name="c", subcore_axis_name="s")
    body = functools.partial(_sc_body, F=F, H=H)
    return pl.kernel(
        body,
        out_type=jax.ShapeDtypeStruct((N, F), jnp.float32),
        mesh=mesh,
        compiler_params=pltpu.CompilerParams(needs_layout_passes=False),
        scratch_types=[
            pltpu.VMEM((NPR, F), jnp.float32),
            pltpu.VMEM((NPR, F), jnp.float32),
            pltpu.VMEM((W, F), jnp.float32),
            pltpu.VMEM((W, F), jnp.float32),
            pltpu.VMEM((W, F), jnp.float32),
            pltpu.VMEM((W, F), jnp.float32),
            pltpu.VMEM((W,), jnp.int32),
            pltpu.VMEM((W,), jnp.int32),
            pltpu.VMEM((W + LANES,), jnp.int32),
            pltpu.VMEM((W + LANES,), jnp.int32),
            pltpu.VMEM((H * NPR + LANES,), jnp.float32),
            pltpu.VMEM((W * H * LANES,), jnp.float32),
            pltpu.VMEM((H * W + LANES,), jnp.float32),
            pltpu.VMEM((F,), jnp.float32),
            pltpu.VMEM((OFFPAD,), jnp.int32),
            pltpu.SemaphoreType.DMA,
            pltpu.SemaphoreType.DMA,
            pltpu.SemaphoreType.DMA,
            pltpu.SemaphoreType.DMA,
        ],
    )(xl, xr, eet, srcs, dsts, offs, attf)


def kernel(x, edge_index, edge_attr, params):
    src, dst = edge_index[0], edge_index[1]
    order = jnp.argsort(dst)
    src_s = jnp.concatenate([src[order], jnp.zeros((EP - E,), jnp.int32)])
    dst_s = jnp.concatenate([dst[order], jnp.zeros((EP - E,), jnp.int32)])
    ea_s = jnp.concatenate(
        [edge_attr[order], jnp.zeros((EP - E, edge_attr.shape[1]), jnp.float32)]
    )
    bounds = jnp.arange(0, N + 1, NPR, dtype=jnp.int32)
    offs = jnp.searchsorted(dst_s[:E], bounds).astype(jnp.int32)
    offs = jnp.concatenate(
        [offs, jnp.full((OFFPAD - NUM_RANGES - 1,), E, jnp.int32)]
    )

    p1, p2 = params["gat1"], params["gat2"]
    f1, f2 = HEADS * HID, HID

    x2 = x.reshape(T * N, D_IN)
    xl1 = _mm_bias(x2, p1["Wl"], p1["bl"], 1000).reshape(T, N, f1)
    xr1 = _mm_bias(x2, p1["Wr"], p1["br"], 1000).reshape(T, N, f1)
    ee1 = _mm_bias(ea_s, p1["We"], jnp.zeros((f1,), jnp.float32), 2000)
    att1 = p1["att"].reshape(-1)
    g1 = [
        _sc_gat(xl1[t], xr1[t], ee1, src_s, dst_s, offs, att1, f1, HEADS)
        for t in range(T)
    ]

    g1all = jnp.concatenate(g1, axis=0)
    xl2, xr2 = _stage2(
        g1all, p1["bias"], p2["Wl"], p2["bl"], p2["Wr"], p2["br"], 1000
    )
    xl2p = jnp.concatenate([xl2[:N], xl2[N:]], axis=1)
    xr2p = jnp.concatenate([xr2[:N], xr2[N:]], axis=1)
    we2p = jnp.concatenate([p2["We"], p2["We"]], axis=1)
    ee2 = _mm_bias(ea_s, we2p, jnp.zeros((2 * f2,), jnp.float32), 2000)
    att2 = jnp.concatenate([p2["att"].reshape(-1)] * 2)
    g2p = _sc_gat(xl2p, xr2p, ee2, src_s, dst_s, offs, att2, 2 * f2, 2)

    s = _gru_mean(g2p[:, :f2], g2p[:, f2:], p2["bias"], params["gru"][0], 1000)
    return _fc(s, params["fc"]["W"], params["fc"]["b"])


# speedup vs baseline: 1.0691x; 1.0009x over previous
import functools

import jax
import jax.numpy as jnp
from jax import lax
from jax.experimental import pallas as pl
from jax.experimental.pallas import tpu as pltpu
from jax.experimental.pallas import tpu_sc as plsc

T, N, D_IN, HID, HEADS = 2, 10000, 128, 64, 8
E = 160000
GRU_H = 256
N_CLASSES = 33

NC, NS, LANES = 2, 16, 16
NW = NC * NS
NPR = 40
NUM_RANGES = N // NPR
RPW = -(-NUM_RANGES // NW)
W = 32
NG = W // LANES
EP = E + 2000
OFFPAD = 272
LEAK = 0.2


def _mm_bias_body(x_ref, w_ref, b_ref, o_ref):
    o_ref[...] = (
        jnp.dot(x_ref[...], w_ref[...], preferred_element_type=jnp.float32)
        + b_ref[...]
    )


def _mm_bias(x, w, b, blk):
    m, k = x.shape
    f = w.shape[1]
    return pl.pallas_call(
        _mm_bias_body,
        grid=(m // blk,),
        in_specs=[
            pl.BlockSpec((blk, k), lambda i: (i, 0)),
            pl.BlockSpec((k, f), lambda i: (0, 0)),
            pl.BlockSpec((1, f), lambda i: (0, 0)),
        ],
        out_specs=pl.BlockSpec((blk, f), lambda i: (i, 0)),
        out_shape=jax.ShapeDtypeStruct((m, f), jnp.float32),
    )(x, w, b[None])


def _elu(v):
    return jnp.where(v > 0, v, jnp.exp(jnp.minimum(v, 0.0)) - 1.0)


def _stage2_body(g_ref, b1_ref, wl_ref, bl_ref, wr_ref, br_ref, xl_ref, xr_ref):
    h = _elu(g_ref[...] + b1_ref[...])
    xl_ref[...] = (
        jnp.dot(h, wl_ref[...], preferred_element_type=jnp.float32) + bl_ref[...]
    )
    xr_ref[...] = (
        jnp.dot(h, wr_ref[...], preferred_element_type=jnp.float32) + br_ref[...]
    )


def _stage2(g, b1, wl, bl, wr, br, blk):
    m, k = g.shape
    f = wl.shape[1]
    return pl.pallas_call(
        _stage2_body,
        grid=(m // blk,),
        in_specs=[
            pl.BlockSpec((blk, k), lambda i: (i, 0)),
            pl.BlockSpec((1, k), lambda i: (0, 0)),
            pl.BlockSpec((k, f), lambda i: (0, 0)),
            pl.BlockSpec((1, f), lambda i: (0, 0)),
            pl.BlockSpec((k, f), lambda i: (0, 0)),
            pl.BlockSpec((1, f), lambda i: (0, 0)),
        ],
        out_specs=[
            pl.BlockSpec((blk, f), lambda i: (i, 0)),
            pl.BlockSpec((blk, f), lambda i: (i, 0)),
        ],
        out_shape=[
            jax.ShapeDtypeStruct((m, f), jnp.float32),
            jax.ShapeDtypeStruct((m, f), jnp.float32),
        ],
    )(g, b1[None], wl, bl[None], wr, br[None])


def _gru_body(o0_ref, o1_ref, b2_ref, wif_ref, bif_ref, whf_ref, bhf_ref,
              wib_ref, bib_ref, whb_ref, bhb_ref, out_ref):
    i = pl.program_id(0)
    s0 = _elu(o0_ref[...] + b2_ref[...])
    s1 = _elu(o1_ref[...] + b2_ref[...])

    def dirstep(xa, xb, wi, bi, wh, bh):
        gia = jnp.dot(xa, wi, preferred_element_type=jnp.float32) + bi
        r = jax.nn.sigmoid(gia[:, :GRU_H] + bh[:, :GRU_H])
        z = jax.nn.sigmoid(gia[:, GRU_H:2 * GRU_H] + bh[:, GRU_H:2 * GRU_H])
        ng = jnp.tanh(gia[:, 2 * GRU_H:] + r * bh[:, 2 * GRU_H:])
        h1 = (1.0 - z) * ng
        gib = jnp.dot(xb, wi, preferred_element_type=jnp.float32) + bi
        gh = jnp.dot(h1, wh, preferred_element_type=jnp.float32) + bh
        r2 = jax.nn.sigmoid(gib[:, :GRU_H] + gh[:, :GRU_H])
        z2 = jax.nn.sigmoid(gib[:, GRU_H:2 * GRU_H] + gh[:, GRU_H:2 * GRU_H])
        n2 = jnp.tanh(gib[:, 2 * GRU_H:] + r2 * gh[:, 2 * GRU_H:])
        return (1.0 - z2) * n2 + z2 * h1

    hf = dirstep(s0, s1, wif_ref[...], bif_ref[...], whf_ref[...], bhf_ref[...])
    hb = dirstep(s1, s0, wib_ref[...], bib_ref[...], whb_ref[...], bhb_ref[...])
    psum = jnp.concatenate(
        [jnp.sum(hf, axis=0, keepdims=True), jnp.sum(hb, axis=0, keepdims=True)],
        axis=1,
    )

    @pl.when(i == 0)
    def _():
        out_ref[...] = jnp.zeros_like(out_ref)

    out_ref[...] += psum


def _gru_mean(o0, o1, b2, gp, blk):
    f = o0.shape[1]
    wif = gp["fwd"]["W_ih"].T
    whf = gp["fwd"]["W_hh"].T
    wib = gp["bwd"]["W_ih"].T
    whb = gp["bwd"]["W_hh"].T
    g3 = 3 * GRU_H
    return pl.pallas_call(
        _gru_body,
        grid=(N // blk,),
        in_specs=[
            pl.BlockSpec((blk, f), lambda i: (i, 0)),
            pl.BlockSpec((blk, f), lambda i: (i, 0)),
            pl.BlockSpec((1, f), lambda i: (0, 0)),
            pl.BlockSpec((f, g3), lambda i: (0, 0)),
            pl.BlockSpec((1, g3), lambda i: (0, 0)),
            pl.BlockSpec((GRU_H, g3), lambda i: (0, 0)),
            pl.BlockSpec((1, g3), lambda i: (0, 0)),
            pl.BlockSpec((f, g3), lambda i: (0, 0)),
            pl.BlockSpec((1, g3), lambda i: (0, 0)),
            pl.BlockSpec((GRU_H, g3), lambda i: (0, 0)),
            pl.BlockSpec((1, g3), lambda i: (0, 0)),
        ],
        out_specs=pl.BlockSpec((1, 2 * GRU_H), lambda i: (0, 0)),
        out_shape=jax.ShapeDtypeStruct((1, 2 * GRU_H), jnp.float32),
    )(o0, o1, b2[None], wif, gp["fwd"]["b_ih"][None], whf,
      gp["fwd"]["b_hh"][None], wib, gp["bwd"]["b_ih"][None], whb,
      gp["bwd"]["b_hh"][None])


def _fc_body(s_ref, w_ref, b_ref, o_ref):
    g = s_ref[...] * (1.0 / N)
    o_ref[...] = (
        jnp.dot(g, w_ref[...], preferred_element_type=jnp.float32) + b_ref[...]
    )


def _fc(s, w, b):
    return pl.pallas_call(
        _fc_body,
        out_shape=jax.ShapeDtypeStruct((1, N_CLASSES), jnp.float32),
    )(s, w, b[None])


def _sc_body(xl, xr, eet, srcs, dsts, offs, attf, out,
             xrbuf, outbuf, xjb0, xjb1, eeb0, eeb1, srb0, srb1, dsb0, dsb1,
             den, parts, exs, attb, offb, s1, s2, s3, sxj, *, F, H):
    xjb, eeb, srb, dsb = (xjb0, xjb1), (eeb0, eeb1), (srb0, srb1), (dsb0, dsb1)
    CPH = (F // H) // LANES
    NCH = F // LANES
    cid = lax.axis_index("c")
    sid = lax.axis_index("s")
    wid = sid * NC + cid
    pltpu.sync_copy(offs, offb)
    pltpu.sync_copy(attf, attb)
    zero = jnp.zeros((LANES,), jnp.float32)
    iot = lax.iota(jnp.int32, LANES)

    def range_body(rr, _):
        rid = wid + rr * NW

        @pl.when(rid < NUM_RANGES)
        def _():
            n0 = rid * NPR
            es = offb[pl.ds(rid, LANES)][0]
            ed = offb[pl.ds(rid + 1, LANES)][0]
            a0 = (es // 8) * 8
            nstrips = (ed - a0 + W - 1) // W

            def zrow(i, _):
                for v in range(NCH):
                    outbuf[i, pl.ds(LANES * v, LANES)] = zero
                return 0

            lax.fori_loop(0, NPR, zrow, 0)
            for j in range((H * NPR + LANES) // LANES):
                den[pl.ds(LANES * j, LANES)] = zero

            pltpu.sync_copy(xr.at[pl.ds(n0, NPR)], xrbuf)

            def load_idx(k, b):
                base = a0 + W * k
                c1 = pltpu.async_copy(srcs.at[pl.ds(base, W)], srb[b], s1)
                c2 = pltpu.async_copy(
                    dsts.at[pl.ds(base, W)], dsb[b].at[pl.ds(0, W)], s2
                )
                c3 = pltpu.async_copy(eet.at[pl.ds(base, W)], eeb[b], s3)
                c1.wait()
                pltpu.async_copy(xl.at[srb[b]], xjb[b], sxj)
                c2.wait()
                c3.wait()

            @pl.when(nstrips > 0)
            def _():
                load_idx(0, 0)

            def compute_strip(k, b):
                base = a0 + W * k
                xjbuf, eebuf, dstb = xjb[b], eeb[b], dsb[b]

                @plsc.parallel_loop(0, W, unroll=2)
                def edge_alpha(e):
                    d = jnp.clip(dstb[pl.ds(e, LANES)][0] - n0, 0, NPR - 1)
                    acc = [zero] * H
                    for v in range(NCH):
                        h = v // CPH
                        sl = pl.ds(LANES * v, LANES)
                        s = xjbuf[e, sl] + xrbuf[d, sl] + eebuf[e, sl]
                        m = jnp.maximum(s, LEAK * s)
                        acc[h] = acc[h] + m * attb[sl]
                    for h in range(H):
                        parts[pl.ds((e * H + h) * LANES, LANES)] = acc[h]

                for g in range(NG):
                    gid = base + g * LANES + iot
                    valid = (gid >= es) & (gid < ed)
                    dstl = jnp.clip(
                        dstb[pl.ds(g * LANES, LANES)] - n0, 0, NPR - 1
                    )
                    for h in range(H):
                        gs = [
                            plsc.load_gather(
                                parts,
                                [(g * LANES + iot) * (H * LANES)
                                 + (h * LANES + l)],
                            )
                            for l in range(LANES)
                        ]
                        while len(gs) > 1:
                            gs = [
                                gs[i] + gs[i + 1]
                                for i in range(0, len(gs) - 1, 2)
                            ] + ([gs[-1]] if len(gs) % 2 else [])
                        exh = jnp.where(valid, jnp.exp(gs[0]), 0.0)
                        plsc.addupdate_scatter(den, [h * NPR + dstl], exh)
                        exs[pl.ds(h * W + g * LANES, LANES)] = exh

                @plsc.parallel_loop(0, W, unroll=2)
                def edge_acc(e):
                    g = base + e

                    @pl.when((g >= es) & (g < ed))
                    def _():
                        d = jnp.clip(dstb[pl.ds(e, LANES)][0] - n0, 0, NPR - 1)
                        for h in range(H):
                            a = exs[pl.ds(h * W + e, LANES)][0]
                            for u in range(CPH):
                                sl = pl.ds(LANES * (h * CPH + u), LANES)
                                plsc.addupdate(
                                    outbuf.at[d, sl], a * xjbuf[e, sl]
                                )

            def outer(ko, _):
                for b in (0, 1):
                    k = 2 * ko + b

                    @pl.when(k < nstrips)
                    def _(k=k, b=b):
                        nb = 1 - b

                        @pl.when(k + 1 < nstrips)
                        def _():
                            base2 = a0 + W * (k + 1)
                            pltpu.async_copy(
                                srcs.at[pl.ds(base2, W)], srb[nb], s1
                            )
                            pltpu.async_copy(
                                dsts.at[pl.ds(base2, W)],
                                dsb[nb].at[pl.ds(0, W)], s2,
                            )
                            pltpu.async_copy(
                                eet.at[pl.ds(base2, W)], eeb[nb], s3
                            )

                        pltpu.make_async_copy(
                            xl.at[pl.ds(0, W)], xjb[b], sxj
                        ).wait()

                        @pl.when(k + 1 < nstrips)
                        def _():
                            pltpu.make_async_copy(
                                srcs.at[pl.ds(0, W)], srb[nb], s1
                            ).wait()
                            pltpu.async_copy(xl.at[srb[nb]], xjb[nb], sxj)
                            pltpu.make_async_copy(
                                dsts.at[pl.ds(0, W)],
                                dsb[nb].at[pl.ds(0, W)], s2,
                            ).wait()
                            pltpu.make_async_copy(
                                eet.at[pl.ds(0, W)], eeb[nb], s3
                            ).wait()

                        compute_strip(k, b)
                return 0

            lax.fori_loop(0, (nstrips + 1) // 2, outer, 0)

            def norm_row(i, _):
                for h in range(H):
                    dsc = den[pl.ds(h * NPR + i, LANES)][0]
                    rv = 1.0 / (jnp.full((LANES,), dsc) + 1e-16)
                    for u in range(CPH):
                        sl = pl.ds(LANES * (h * CPH + u), LANES)
                        outbuf[i, sl] = outbuf[i, sl] * rv
                return 0

            lax.fori_loop(0, NPR, norm_row, 0)
            pltpu.sync_copy(outbuf, out.at[pl.ds(n0, NPR)])
        return 0

    lax.fori_loop(0, RPW, range_body, 0)


def _sc_gat(xl, xr, eet, srcs, dsts, offs, attf, F, H):
    mesh = plsc.VectorSubcoreMesh(core_axis_name="c", subcore_axis_name="s")
    body = functools.partial(_sc_body, F=F, H=H)
    return pl.kernel(
        body,
        out_type=jax.ShapeDtypeStruct((N, F), jnp.float32),
        mesh=mesh,
        compiler_params=pltpu.CompilerParams(needs_layout_passes=False),
        scratch_types=[
            pltpu.VMEM((NPR, F), jnp.float32),
            pltpu.VMEM((NPR, F), jnp.float32),
            pltpu.VMEM((W, F), jnp.float32),
            pltpu.VMEM((W, F), jnp.float32),
            pltpu.VMEM((W, F), jnp.float32),
            pltpu.VMEM((W, F), jnp.float32),
            pltpu.VMEM((W,), jnp.int32),
            pltpu.VMEM((W,), jnp.int32),
            pltpu.VMEM((W + LANES,), jnp.int32),
            pltpu.VMEM((W + LANES,), jnp.int32),
            pltpu.VMEM((H * NPR + LANES,), jnp.float32),
            pltpu.VMEM((W * H * LANES,), jnp.float32),
            pltpu.VMEM((H * W + LANES,), jnp.float32),
            pltpu.VMEM((F,), jnp.float32),
            pltpu.VMEM((OFFPAD,), jnp.int32),
            pltpu.SemaphoreType.DMA,
            pltpu.SemaphoreType.DMA,
            pltpu.SemaphoreType.DMA,
            pltpu.SemaphoreType.DMA,
        ],
    )(xl, xr, eet, srcs, dsts, offs, attf)


def kernel(x, edge_index, edge_attr, params):
    src, dst = edge_index[0], edge_index[1]
    order = jnp.argsort(dst)
    src_s = jnp.concatenate([src[order], jnp.zeros((EP - E,), jnp.int32)])
    dst_s = jnp.concatenate([dst[order], jnp.zeros((EP - E,), jnp.int32)])
    ea_s = jnp.concatenate(
        [edge_attr[order], jnp.zeros((EP - E, edge_attr.shape[1]), jnp.float32)]
    )
    bounds = jnp.arange(0, N + 1, NPR, dtype=jnp.int32)
    offs = jnp.searchsorted(dst_s[:E], bounds).astype(jnp.int32)
    offs = jnp.concatenate(
        [offs, jnp.full((OFFPAD - NUM_RANGES - 1,), E, jnp.int32)]
    )

    p1, p2 = params["gat1"], params["gat2"]
    f1, f2 = HEADS * HID, HID

    x2 = x.reshape(T * N, D_IN)
    xl1 = _mm_bias(x2, p1["Wl"], p1["bl"], 1000).reshape(T, N, f1)
    xr1 = _mm_bias(x2, p1["Wr"], p1["br"], 1000).reshape(T, N, f1)
    ee1 = _mm_bias(ea_s, p1["We"], jnp.zeros((f1,), jnp.float32), 2000)
    att1 = p1["att"].reshape(-1)
    g1 = [
        _sc_gat(xl1[t], xr1[t], ee1, src_s, dst_s, offs, att1, f1, HEADS)
        for t in range(T)
    ]

    g1all = jnp.concatenate(g1, axis=0)
    xl2, xr2 = _stage2(
        g1all, p1["bias"], p2["Wl"], p2["bl"], p2["Wr"], p2["br"], 1000
    )
    xl2p = jnp.concatenate([xl2[:N], xl2[N:]], axis=1)
    xr2p = jnp.concatenate([xr2[:N], xr2[N:]], axis=1)
    we2p = jnp.concatenate([p2["We"], p2["We"]], axis=1)
    ee2 = _mm_bias(ea_s, we2p, jnp.zeros((2 * f2,), jnp.float32), 2000)
    att2 = jnp.concatenate([p2["att"].reshape(-1)] * 2)
    g2p = _sc_gat(xl2p, xr2p, ee2, src_s, dst_s, offs, att2, 2 * f2, 2)

    s = _gru_mean(g2p[:, :f2], g2p[:, f2:], p2["bias"], params["gru"][0], 1000)
    return _fc(s, params["fc"]["W"], params["fc"]["b"])
